# Initial kernel scaffold; baseline (speedup 1.0000x reference)
import jax
import jax.numpy as jnp
from jax.experimental import pallas as pl


def kernel(generated, known, mask):
    raise NotImplementedError("write your pallas kernel here")


# speedup vs baseline: 8.2676x; 8.2676x over previous
import jax
import jax.numpy as jnp
from jax.experimental import pallas as pl

BR = 128
NP = 1024
NB = NP // BR

NEG_INF = float("-inf")


def _sim_top2_kernel(pb_ref, kf_ref, pf_ref, maskrow_ref,
                     npq_ref, nk_row_ref, np_row_ref,
                     v0a_ref, v0b_ref, v1a_ref, v1b_ref,
                     i0a_ref, i0b_ref, i1a_ref, i1b_ref):
    pb = pb_ref[...]
    kf = kf_ref[...]
    pf = pf_ref[...]
    maskrow = maskrow_ref[...]
    npq = npq_ref[...]
    nk_row = nk_row_ref[...]
    np_row = np_row_ref[...]

    dn = (((1,), (1,)), ((), ()))
    s0 = jax.lax.dot_general(pb.astype(jnp.bfloat16), kf.astype(jnp.bfloat16),
                             dn, preferred_element_type=jnp.float32)
    s1 = jax.lax.dot_general(pb.astype(jnp.bfloat16), pf.astype(jnp.bfloat16),
                             dn, preferred_element_type=jnp.float32)

    ci = jax.lax.broadcasted_iota(jnp.int32, (BR, NP), 1)

    def top2(scores):
        m1 = jnp.max(scores, axis=1, keepdims=True)
        im1 = jnp.min(jnp.where(scores == m1, ci, NP + 1),
                      axis=1, keepdims=True)
        scores2 = jnp.where(ci == im1, NEG_INF, scores)
        m2 = jnp.max(scores2, axis=1, keepdims=True)
        im2 = jnp.min(jnp.where(scores2 == m2, ci, NP + 1),
                      axis=1, keepdims=True)
        return m1, m2, im1.astype(jnp.int32), im2.astype(jnp.int32)

    is_masked_col = maskrow > 0.5
    cos0 = jnp.where(is_masked_col, NEG_INF, s0 / jnp.sqrt(npq * nk_row))
    cos1 = jnp.where(is_masked_col, s1 / jnp.sqrt(npq * np_row), NEG_INF)

    v0a, v0b, i0a, i0b = top2(cos0)
    v1a, v1b, i1a, i1b = top2(cos1)

    v0a_ref[...] = v0a
    v0b_ref[...] = v0b
    v1a_ref[...] = v1a
    v1b_ref[...] = v1b
    i0a_ref[...] = i0a
    i0b_ref[...] = i0b
    i1a_ref[...] = i1a
    i1b_ref[...] = i1b


def _gather_kernel(i0a_ref, i0b_ref, i1a_ref, i1b_ref,
                   v0a_ref, v0b_ref, v1a_ref, v1b_ref,
                   mf_ref, kf_ref, pf_ref, out_ref):
    rb = pl.program_id(0)
    mf = mf_ref[...]
    n_masked = jnp.sum(mf)

    e0 = jnp.exp(jnp.sum(v0a_ref[...] * mf) / n_masked)
    e1 = jnp.exp(jnp.sum(v0b_ref[...] * mf) / n_masked)
    e2 = jnp.exp(jnp.sum(v1a_ref[...] * mf) / n_masked)
    e3 = jnp.exp(jnp.sum(v1b_ref[...] * mf) / n_masked)
    denom = e0 + e1 + e2 + e3
    w0 = e0 / denom
    w1 = e1 / denom
    w2 = e2 / denom
    w3 = e3 / denom

    mrow = mf_ref[pl.ds(rb * BR, BR), :]
    ci = jax.lax.broadcasted_iota(jnp.int32, (BR, NP), 1)
    gr = rb * BR + jax.lax.broadcasted_iota(jnp.int32, (BR, NP), 0)
    row0 = gr == 0
    col0 = jnp.where(ci == 0, 1.0, 0.0)
    masked_row = mrow > 0.5

    def onehot(idx_ref):
        idxb = idx_ref[...]
        base = jnp.where(masked_row & (ci == idxb), 1.0, 0.0)
        oh = jnp.where(row0, jnp.maximum(base, col0), base)
        return oh.astype(jnp.bfloat16)

    kf = kf_ref[...].astype(jnp.bfloat16)
    pf = pf_ref[...].astype(jnp.bfloat16)
    m0 = jnp.dot(onehot(i0a_ref), kf, preferred_element_type=jnp.float32)
    m1 = jnp.dot(onehot(i0b_ref), kf, preferred_element_type=jnp.float32)
    m2 = jnp.dot(onehot(i1a_ref), pf, preferred_element_type=jnp.float32)
    m3 = jnp.dot(onehot(i1b_ref), pf, preferred_element_type=jnp.float32)
    out_ref[...] = ((m0 * w0 + m1 * w1) + m2 * w2) + m3 * w3


@jax.jit
def _run(generated, known, mask):
    c = generated.shape[1]
    pm = generated.reshape(c, NP).T
    km = known.reshape(c, NP).T
    maskc = mask.reshape(NP, 1)
    maskrow = mask.reshape(1, NP)

    np_row = jnp.einsum('bij,bij->bi', pm[None], pm[None])
    nk_row = jnp.einsum('bij,bij->bi', km[None], km[None])
    np_col = np_row.reshape(NP, 1)

    colvec = jax.ShapeDtypeStruct((NP, 1), jnp.float32)
    colvec_i = jax.ShapeDtypeStruct((NP, 1), jnp.int32)

    blk_rows = pl.BlockSpec((BR, c), lambda i: (i, 0))
    blk_full = pl.BlockSpec((NP, c), lambda i: (0, 0))
    blk_mrow = pl.BlockSpec((1, NP), lambda i: (0, 0))
    blk_cvec_full = pl.BlockSpec((NP, 1), lambda i: (0, 0))
    blk_cvec = pl.BlockSpec((BR, 1), lambda i: (i, 0))

    v0a, v0b, v1a, v1b, i0a, i0b, i1a, i1b = pl.pallas_call(
        _sim_top2_kernel,
        grid=(NB,),
        in_specs=[blk_rows, blk_full, blk_full, blk_mrow,
                  blk_cvec, blk_mrow, blk_mrow],
        out_specs=[blk_cvec] * 8,
        out_shape=[colvec] * 4 + [colvec_i] * 4,
    )(pm, km, pm, maskrow, np_col, nk_row, np_row)

    rtn = pl.pallas_call(
        _gather_kernel,
        grid=(NB,),
        in_specs=[blk_cvec] * 4 + [blk_cvec_full] * 5 + [blk_full, blk_full],
        out_specs=blk_rows,
        out_shape=jax.ShapeDtypeStruct((NP, c), jnp.float32),
    )(i0a, i0b, i1a, i1b, v0a, v0b, v1a, v1b, maskc, km, pm)

    rtn = jnp.transpose(rtn.reshape(1, 32, 32, c), (0, 3, 1, 2))
    return jnp.concatenate([generated, known, rtn], axis=1)


def kernel(generated, known, mask):
    return _run(generated, known, mask)
